# baseline (speedup 1.0000x reference)
import jax
import jax.numpy as jnp
from jax import lax
from jax.experimental import pallas as pl
from jax.experimental.pallas import tpu as pltpu
from jax.experimental.pallas import tpu_sc as plsc

_ROWS = 16384
_COLS = 4096
_BLOCK_R = 512
_LANES = 16


def _copy_body(x_ref, o_ref):
    o_ref[...] = x_ref[...]


def _tc_copy(x):
    return pl.pallas_call(
        _copy_body,
        grid=(_ROWS // _BLOCK_R,),
        in_specs=[pl.BlockSpec((_BLOCK_R, _COLS), lambda i: (i, 0))],
        out_specs=pl.BlockSpec((_BLOCK_R, _COLS), lambda i: (i, 0)),
        out_shape=jax.ShapeDtypeStruct((_ROWS, _COLS), jnp.float32),
        compiler_params=pltpu.CompilerParams(
            dimension_semantics=("arbitrary",),
        ),
    )(x)


def _sc_patch_body(i1_hbm, i2_hbm, v_hbm, out_hbm, ridx_v, cidx_v, val_v,
                   row_v, sem):
    wid = lax.axis_index("s") * 2 + lax.axis_index("c")

    @pl.when(wid == 0)
    def _():
        pltpu.sync_copy(i1_hbm, ridx_v)
        pltpu.sync_copy(i2_hbm, cidx_v)
        pltpu.sync_copy(v_hbm, val_v)
        pltpu.async_copy(out_hbm.at[ridx_v], row_v, sem).wait()
        lanes = lax.iota(jnp.int32, _LANES)
        mask = lanes == 0
        zeros = lanes * 0
        plsc.store_scatter(row_v, [zeros, cidx_v[...]], val_v[...], mask=mask)
        pltpu.async_copy(row_v, out_hbm.at[ridx_v], sem).wait()


_sc_patch = pl.kernel(
    _sc_patch_body,
    out_type=(),
    mesh=plsc.VectorSubcoreMesh(core_axis_name="c", subcore_axis_name="s"),
    compiler_params=pltpu.CompilerParams(needs_layout_passes=False),
    scratch_types=[
        pltpu.VMEM((1,), jnp.int32),
        pltpu.VMEM((_LANES,), jnp.int32),
        pltpu.VMEM((_LANES,), jnp.float32),
        pltpu.VMEM((1, _COLS), jnp.float32),
        pltpu.SemaphoreType.DMA,
    ],
)


def kernel(input, index1, index2, value):
    i1 = index1.astype(jnp.int32)
    i2 = jnp.broadcast_to(index2.astype(jnp.int32), (_LANES,))
    v = jnp.broadcast_to(value.astype(jnp.float32), (_LANES,))

    copied = _tc_copy(input)
    out_ref = jax.new_ref(copied)
    _sc_patch(i1, i2, v, out_ref)
    return jax.freeze(out_ref)


# speedup vs baseline: 1.0089x; 1.0089x over previous
import jax
import jax.numpy as jnp
from jax import lax
from jax.experimental import pallas as pl
from jax.experimental.pallas import tpu as pltpu
from jax.experimental.pallas import tpu_sc as plsc

_ROWS = 16384
_COLS = 4096
_BLOCK_R = 512
_LANES = 16


def _copy_body(x_ref, o_ref):
    o_ref[...] = x_ref[...]


def _tc_copy(x):
    return pl.pallas_call(
        _copy_body,
        grid=(_ROWS // _BLOCK_R,),
        in_specs=[pl.BlockSpec((_BLOCK_R, _COLS), lambda i: (i, 0))],
        out_specs=pl.BlockSpec((_BLOCK_R, _COLS), lambda i: (i, 0)),
        out_shape=jax.ShapeDtypeStruct((_ROWS, _COLS), jnp.float32),
        compiler_params=pltpu.CompilerParams(
            dimension_semantics=("arbitrary",),
        ),
    )(x)


def _sc_row_body(i1_hbm, i2_hbm, v_hbm, x_hbm, row_out_hbm, ridx_v, cidx_v,
                 val_v, row_v, sem):
    wid = lax.axis_index("s") * 2 + lax.axis_index("c")

    @pl.when(wid == 0)
    def _():
        pltpu.sync_copy(i1_hbm, ridx_v)
        pltpu.sync_copy(i2_hbm, cidx_v)
        pltpu.sync_copy(v_hbm, val_v)
        pltpu.async_copy(x_hbm.at[ridx_v], row_v, sem).wait()
        lanes = lax.iota(jnp.int32, _LANES)
        mask = lanes == 0
        zeros = lanes * 0
        plsc.store_scatter(row_v, [zeros, cidx_v[...]], val_v[...], mask=mask)
        pltpu.sync_copy(row_v, row_out_hbm)


_sc_make_row = pl.kernel(
    _sc_row_body,
    out_type=jax.ShapeDtypeStruct((1, _COLS), jnp.float32),
    mesh=plsc.VectorSubcoreMesh(core_axis_name="c", subcore_axis_name="s"),
    compiler_params=pltpu.CompilerParams(needs_layout_passes=False),
    scratch_types=[
        pltpu.VMEM((1,), jnp.int32),
        pltpu.VMEM((_LANES,), jnp.int32),
        pltpu.VMEM((_LANES,), jnp.float32),
        pltpu.VMEM((1, _COLS), jnp.float32),
        pltpu.SemaphoreType.DMA,
    ],
)


def _row_write_body(i1_ref, copied_ref, row_ref, o_ref, sem):
    row = i1_ref[0]
    put = pltpu.make_async_copy(row_ref, o_ref.at[pl.ds(row, 1), :], sem)
    put.start()
    put.wait()


def _tc_row_write(copied, row, i1):
    return pl.pallas_call(
        _row_write_body,
        in_specs=[
            pl.BlockSpec(memory_space=pltpu.SMEM),
            pl.BlockSpec(memory_space=pl.ANY),
            pl.BlockSpec(memory_space=pltpu.VMEM),
        ],
        out_specs=pl.BlockSpec(memory_space=pl.ANY),
        out_shape=jax.ShapeDtypeStruct((_ROWS, _COLS), jnp.float32),
        input_output_aliases={1: 0},
        scratch_shapes=[pltpu.SemaphoreType.DMA],
    )(i1, copied, row)


def kernel(input, index1, index2, value):
    i1 = index1.astype(jnp.int32)
    i2 = jnp.broadcast_to(index2.astype(jnp.int32), (_LANES,))
    v = jnp.broadcast_to(value.astype(jnp.float32), (_LANES,))

    row = _sc_make_row(i1, i2, v, input)
    copied = _tc_copy(input)
    return _tc_row_write(copied, row, i1)


# speedup vs baseline: 1.0259x; 1.0169x over previous
import jax
import jax.numpy as jnp
from jax import lax
from jax.experimental import pallas as pl
from jax.experimental.pallas import tpu as pltpu
from jax.experimental.pallas import tpu_sc as plsc

_ROWS = 16384
_COLS = 4096
_BLOCK_R = 512
_LANES = 16


def _copy_body(x_ref, o_ref):
    o_ref[...] = x_ref[...]


def _tc_copy(x):
    return pl.pallas_call(
        _copy_body,
        grid=(_ROWS // _BLOCK_R,),
        in_specs=[pl.BlockSpec((_BLOCK_R, _COLS), lambda i: (i, 0))],
        out_specs=pl.BlockSpec((_BLOCK_R, _COLS), lambda i: (i, 0)),
        out_shape=jax.ShapeDtypeStruct((_ROWS, _COLS), jnp.float32),
        compiler_params=pltpu.CompilerParams(
            dimension_semantics=("arbitrary",),
        ),
    )(x)


def _sc_row_body(i1_hbm, i2_hbm, v_hbm, x_hbm, row_out_hbm, ridx_v, cidx_v,
                 val_v, row_v, sem):
    wid = lax.axis_index("s")

    @pl.when(wid == 0)
    def _():
        lanes = lax.iota(jnp.int32, _LANES)
        cidx_v[...] = lanes * 0
        val_v[...] = lanes * 0.0
        pltpu.sync_copy(i1_hbm, ridx_v)
        pltpu.sync_copy(i2_hbm, cidx_v.at[pl.ds(0, 1)])
        pltpu.sync_copy(v_hbm, val_v.at[pl.ds(0, 1)])
        pltpu.async_copy(x_hbm.at[ridx_v], row_v, sem).wait()
        mask = lanes == 0
        zeros = lanes * 0
        plsc.store_scatter(row_v, [zeros, cidx_v[...]], val_v[...], mask=mask)
        pltpu.sync_copy(row_v, row_out_hbm)


_sc_make_row = pl.kernel(
    _sc_row_body,
    out_type=jax.ShapeDtypeStruct((1, _COLS), jnp.float32),
    mesh=plsc.VectorSubcoreMesh(
        core_axis_name="c", subcore_axis_name="s", num_cores=1
    ),
    compiler_params=pltpu.CompilerParams(needs_layout_passes=False),
    scratch_types=[
        pltpu.VMEM((1,), jnp.int32),
        pltpu.VMEM((_LANES,), jnp.int32),
        pltpu.VMEM((_LANES,), jnp.float32),
        pltpu.VMEM((1, _COLS), jnp.float32),
        pltpu.SemaphoreType.DMA,
    ],
)


def _row_write_body(i1_ref, copied_ref, row_ref, o_ref, sem):
    row = i1_ref[0]
    put = pltpu.make_async_copy(row_ref, o_ref.at[pl.ds(row, 1), :], sem)
    put.start()
    put.wait()


def _tc_row_write(copied, row, i1):
    return pl.pallas_call(
        _row_write_body,
        in_specs=[
            pl.BlockSpec(memory_space=pltpu.SMEM),
            pl.BlockSpec(memory_space=pl.ANY),
            pl.BlockSpec(memory_space=pltpu.VMEM),
        ],
        out_specs=pl.BlockSpec(memory_space=pl.ANY),
        out_shape=jax.ShapeDtypeStruct((_ROWS, _COLS), jnp.float32),
        input_output_aliases={1: 0},
        scratch_shapes=[pltpu.SemaphoreType.DMA],
    )(i1, copied, row)


def kernel(input, index1, index2, value):
    i1 = index1.astype(jnp.int32)
    i2 = index2.astype(jnp.int32)
    v = value.astype(jnp.float32)

    row = _sc_make_row(i1, i2, v, input)
    copied = _tc_copy(input)
    return _tc_row_write(copied, row, i1)


# speedup vs baseline: 1.0265x; 1.0005x over previous
import jax
import jax.numpy as jnp
from jax import lax
from jax.experimental import pallas as pl
from jax.experimental.pallas import tpu as pltpu
from jax.experimental.pallas import tpu_sc as plsc

_ROWS = 16384
_COLS = 4096
_BLOCK_R = 512
_LANES = 16


def _copy_body(x_ref, o_ref):
    o_ref[...] = x_ref[...]


def _tc_copy(x):
    return pl.pallas_call(
        _copy_body,
        grid=(_ROWS // _BLOCK_R,),
        in_specs=[pl.BlockSpec((_BLOCK_R, _COLS), lambda i: (i, 0))],
        out_specs=pl.BlockSpec((_BLOCK_R, _COLS), lambda i: (i, 0)),
        out_shape=jax.ShapeDtypeStruct((_ROWS, _COLS), jnp.float32),
        compiler_params=pltpu.CompilerParams(
            dimension_semantics=("arbitrary",),
        ),
    )(x)


def _sc_row_body(i1_hbm, i2_hbm, v_hbm, x_hbm, row_out_hbm, ridx_v, cidx_v,
                 val_v, row_v, sem):
    wid = lax.axis_index("s")

    @pl.when(wid == 0)
    def _():
        lanes = lax.iota(jnp.int32, _LANES)
        cidx_v[...] = lanes * 0
        val_v[...] = lanes * 0.0
        pltpu.sync_copy(i1_hbm, ridx_v)
        pltpu.sync_copy(i2_hbm, cidx_v.at[pl.ds(0, 1)])
        pltpu.sync_copy(v_hbm, val_v.at[pl.ds(0, 1)])
        pltpu.async_copy(x_hbm.at[ridx_v], row_v, sem).wait()
        mask = lanes == 0
        zeros = lanes * 0
        plsc.store_scatter(row_v, [zeros, cidx_v[...]], val_v[...], mask=mask)
        pltpu.sync_copy(row_v, row_out_hbm)


_sc_make_row = pl.kernel(
    _sc_row_body,
    out_type=jax.ShapeDtypeStruct((1, _COLS), jnp.float32),
    mesh=plsc.VectorSubcoreMesh(
        core_axis_name="c", subcore_axis_name="s", num_cores=1
    ),
    compiler_params=pltpu.CompilerParams(
        needs_layout_passes=False, skip_device_barrier=True
    ),
    scratch_types=[
        pltpu.VMEM((1,), jnp.int32),
        pltpu.VMEM((_LANES,), jnp.int32),
        pltpu.VMEM((_LANES,), jnp.float32),
        pltpu.VMEM((1, _COLS), jnp.float32),
        pltpu.SemaphoreType.DMA,
    ],
)


def _row_write_body(i1_ref, copied_ref, row_ref, o_ref, sem):
    row = i1_ref[0]
    put = pltpu.make_async_copy(row_ref, o_ref.at[pl.ds(row, 1), :], sem)
    put.start()
    put.wait()


def _tc_row_write(copied, row, i1):
    return pl.pallas_call(
        _row_write_body,
        in_specs=[
            pl.BlockSpec(memory_space=pltpu.SMEM),
            pl.BlockSpec(memory_space=pl.ANY),
            pl.BlockSpec(memory_space=pltpu.VMEM),
        ],
        out_specs=pl.BlockSpec(memory_space=pl.ANY),
        out_shape=jax.ShapeDtypeStruct((_ROWS, _COLS), jnp.float32),
        input_output_aliases={1: 0},
        scratch_shapes=[pltpu.SemaphoreType.DMA],
    )(i1, copied, row)


def kernel(input, index1, index2, value):
    i1 = index1.astype(jnp.int32)
    i2 = index2.astype(jnp.int32)
    v = value.astype(jnp.float32)

    row = _sc_make_row(i1, i2, v, input)
    copied = _tc_copy(input)
    return _tc_row_write(copied, row, i1)
